# baseline (speedup 1.0000x reference)
import functools

import jax
import jax.numpy as jnp
from jax import lax
from jax.experimental import pallas as pl
from jax.experimental.pallas import tpu as pltpu
from jax.experimental.pallas import tpu_sc as plsc

NUM_MATRIX = 8
H = 1024
W = 128
B = 16384

_INFO = plsc.get_sparse_core_info()
NC = _INFO.num_cores
NS = _INFO.num_subcores
L = _INFO.num_lanes
NW = NC * NS
BPW = B // NW
CH = 128
NCH = BPW // CH

_mesh = plsc.VectorSubcoreMesh(core_axis_name="c", subcore_axis_name="s")


@functools.partial(
    pl.kernel,
    mesh=_mesh,
    out_type=jax.ShapeDtypeStruct((B, W), jnp.float32),
    scratch_types=[
        pltpu.VMEM((BPW,), jnp.int32),
        pltpu.VMEM((BPW,), jnp.int32),
        pltpu.VMEM((NCH, CH), jnp.int32),
        pltpu.VMEM((BPW, W), jnp.float32),
        pltpu.SemaphoreType.DMA,
        pltpu.SemaphoreType.DMA,
    ],
)
def _uvnet_gather(x1_hbm, x2_hbm, table_hbm, out_hbm,
                  x1_v, x2_v, idx_v, rows_v, gsem, wsem):
    wid = lax.axis_index("s") * NC + lax.axis_index("c")
    base = wid * BPW
    pltpu.sync_copy(x1_hbm.at[pl.ds(base, BPW)], x1_v)
    pltpu.sync_copy(x2_hbm.at[pl.ds(base, BPW)], x2_v)
    gathers = []
    for j in range(NCH):
        for t in range(CH // L):
            s = j * CH + t * L
            idx_v[j, pl.ds(t * L, L)] = x1_v[pl.ds(s, L)] * H + x2_v[pl.ds(s, L)]
        gathers.append(pltpu.async_copy(table_hbm.at[idx_v.at[j]],
                                        rows_v.at[pl.ds(j * CH, CH)], gsem))
    writes = []
    for j in range(NCH):
        gathers[j].wait()
        writes.append(pltpu.async_copy(rows_v.at[pl.ds(j * CH, CH)],
                                       out_hbm.at[pl.ds(base + j * CH, CH)],
                                       wsem))
    for c in writes:
        c.wait()


def kernel(x1, x2, matrices):
    table = matrices.reshape(NUM_MATRIX * H, W)
    out = _uvnet_gather(x1.astype(jnp.int32), x2.astype(jnp.int32), table)
    return out.reshape(1, B, 1, W)


# speedup vs baseline: 1.0416x; 1.0416x over previous
import functools

import jax
import jax.numpy as jnp
from jax import lax
from jax.experimental import pallas as pl
from jax.experimental.pallas import tpu as pltpu
from jax.experimental.pallas import tpu_sc as plsc

NUM_MATRIX = 8
H = 1024
W = 128
B = 16384

_INFO = plsc.get_sparse_core_info()
NC = _INFO.num_cores
NS = _INFO.num_subcores
L = _INFO.num_lanes
NW = NC * NS
BPW = B // NW
CH = 128
NCH = BPW // CH

_mesh = plsc.VectorSubcoreMesh(core_axis_name="c", subcore_axis_name="s")


@functools.partial(
    pl.kernel,
    mesh=_mesh,
    out_type=jax.ShapeDtypeStruct((B, W), jnp.float32),
    scratch_types=[
        pltpu.VMEM((BPW,), jnp.int32),
        pltpu.VMEM((BPW,), jnp.int32),
        pltpu.VMEM((NCH, CH), jnp.int32),
        pltpu.VMEM((BPW, W), jnp.float32),
        pltpu.SemaphoreType.DMA,
        pltpu.SemaphoreType.DMA,
    ],
)
def _uvnet_gather(x1_hbm, x2_hbm, table_hbm, out_hbm,
                  x1_v, x2_v, idx_v, rows_v, gsem, wsem):
    wid = lax.axis_index("s") * NC + lax.axis_index("c")
    base = wid * BPW
    c1 = pltpu.async_copy(x1_hbm.at[pl.ds(base, BPW)], x1_v, wsem)
    c2 = pltpu.async_copy(x2_hbm.at[pl.ds(base, BPW)], x2_v, wsem)
    c1.wait()
    c2.wait()
    gathers = []
    for j in range(NCH):
        for t in range(CH // L):
            s = j * CH + t * L
            idx_v[j, pl.ds(t * L, L)] = x1_v[pl.ds(s, L)] * H + x2_v[pl.ds(s, L)]
        gathers.append(pltpu.async_copy(table_hbm.at[idx_v.at[j]],
                                        rows_v.at[pl.ds(j * CH, CH)], gsem))
    for c in gathers:
        c.wait()
    pltpu.sync_copy(rows_v, out_hbm.at[pl.ds(base, BPW)])


def kernel(x1, x2, matrices):
    table = matrices.reshape(NUM_MATRIX * H, W)
    out = _uvnet_gather(x1.astype(jnp.int32), x2.astype(jnp.int32), table)
    return out.reshape(1, B, 1, W)
